# baseline (speedup 1.0000x reference)
import functools

import jax
import jax.numpy as jnp
from jax import lax
from jax.experimental import pallas as pl
from jax.experimental.pallas import tpu as pltpu
from jax.experimental.pallas import tpu_sc as plsc

NWORK = 32
GRP = 128


def _argmin_body(cz_ref, pzt_ref, out_ref, rmin, ridx, hmin, hidx):
    b = pl.program_id(0)
    j = pl.program_id(1)
    nch = pl.num_programs(1)
    half = nch // 2
    nc = pzt_ref.shape[2]
    n_total = nc * nch
    m = cz_ref.shape[1]

    cz = cz_ref[0]
    pzt = pzt_ref[0]
    c2 = jnp.sum(cz * cz, axis=1, keepdims=True)
    p2 = jnp.sum(pzt * pzt, axis=0, keepdims=True)
    cp = jnp.dot(cz, pzt, preferred_element_type=jnp.float32)
    d2 = (c2 + p2) - 2.0 * cp

    cmin = jnp.min(d2, axis=1, keepdims=True)
    lane = lax.broadcasted_iota(jnp.int32, d2.shape, 1)
    big = jnp.int32(2**30)
    cidx = jnp.min(jnp.where(d2 == cmin, lane, big), axis=1, keepdims=True)
    cidx = cidx + j * nc

    cmin_b = jnp.broadcast_to(cmin, (m, 128))
    cidx_b = jnp.broadcast_to(cidx, (m, 128))

    @pl.when((j == 0) | (j == half))
    def _():
        rmin[...] = cmin_b
        ridx[...] = cidx_b

    @pl.when((j != 0) & (j != half))
    def _():
        better = cmin_b < rmin[...]
        rmin[...] = jnp.where(better, cmin_b, rmin[...])
        ridx[...] = jnp.where(better, cidx_b, ridx[...])

    @pl.when(j == half - 1)
    def _():
        hmin[...] = rmin[...]
        hidx[...] = ridx[...]

    @pl.when(j == nch - 1)
    def _():
        h1b = hmin[...].astype(jnp.bfloat16).astype(jnp.float32)
        steal = rmin[...] < h1b
        nn = jnp.where(steal, ridx[...], hidx[...])
        out_ref[0] = nn[:, 0:1] + b * n_total


def _nn_argmin(centers_xyz, pzt, nc=512):
    bsz, m, _ = centers_xyz.shape
    n = pzt.shape[2]
    grid = (bsz, n // nc)
    return pl.pallas_call(
        _argmin_body,
        grid=grid,
        in_specs=[
            pl.BlockSpec((1, m, 3), lambda b, j: (b, 0, 0)),
            pl.BlockSpec((1, 3, nc), lambda b, j: (b, 0, j)),
        ],
        out_specs=pl.BlockSpec((1, m, 1), lambda b, j: (b, 0, 0)),
        out_shape=jax.ShapeDtypeStruct((bsz, m, 1), jnp.int32),
        scratch_shapes=[
            pltpu.VMEM((m, 128), jnp.float32),
            pltpu.VMEM((m, 128), jnp.int32),
            pltpu.VMEM((m, 128), jnp.float32),
            pltpu.VMEM((m, 128), jnp.int32),
        ],
        compiler_params=pltpu.CompilerParams(
            dimension_semantics=("arbitrary", "arbitrary")),
    )(centers_xyz, pzt)


def _make_sc_gather_pair(r, f, chunk):
    ch = r // NWORK
    nt = ch // chunk
    ng = chunk // GRP
    mesh = plsc.VectorSubcoreMesh(core_axis_name="c", subcore_axis_name="s")

    @functools.partial(
        pl.kernel,
        out_type=(jax.ShapeDtypeStruct((r, f), jnp.float32),
                  jax.ShapeDtypeStruct((r, 16), jnp.float32)),
        mesh=mesh,
        scratch_types=[
            pltpu.VMEM((ch // GRP, GRP), jnp.int32),
            pltpu.VMEM((chunk, f), jnp.float32),
            pltpu.VMEM((chunk, 16), jnp.float32),
            pltpu.SemaphoreType.DMA,
            pltpu.SemaphoreType.DMA,
        ],
        compiler_params=pltpu.CompilerParams(use_tc_tiling_on_sc=False),
    )
    def k(idx_hbm, feats_hbm, xyz16_hbm, nf_hbm, nx_hbm,
          idx_v, fbuf, xbuf, semf, semx):
        wid = lax.axis_index("s") * 2 + lax.axis_index("c")
        base = wid * ch
        pltpu.sync_copy(idx_hbm.at[pl.ds(wid * (ch // GRP), ch // GRP)], idx_v)
        for t in range(nt):
            cps = []
            for g in range(ng):
                row = t * ng + g
                cps.append(pltpu.async_copy(
                    feats_hbm.at[idx_v.at[row]],
                    fbuf.at[pl.ds(g * GRP, GRP)], semf))
                cps.append(pltpu.async_copy(
                    xyz16_hbm.at[idx_v.at[row]],
                    xbuf.at[pl.ds(g * GRP, GRP)], semx))
            for c in cps:
                c.wait()
            pltpu.sync_copy(fbuf, nf_hbm.at[pl.ds(base + t * chunk, chunk)])
            pltpu.sync_copy(xbuf, nx_hbm.at[pl.ds(base + t * chunk, chunk)])

    return k


def _make_sc_gather_center(g, f):
    ch = g // NWORK
    assert ch == GRP
    mesh = plsc.VectorSubcoreMesh(core_axis_name="c", subcore_axis_name="s")

    @functools.partial(
        pl.kernel,
        out_type=jax.ShapeDtypeStruct((g, f), jnp.float32),
        mesh=mesh,
        scratch_types=[
            pltpu.VMEM((GRP,), jnp.int32),
            pltpu.VMEM((GRP, f), jnp.float32),
            pltpu.SemaphoreType.DMA,
        ],
        compiler_params=pltpu.CompilerParams(use_tc_tiling_on_sc=False),
    )
    def k(idx_hbm, feats_hbm, cf_hbm, idx_v, fbuf, sem):
        wid = lax.axis_index("s") * 2 + lax.axis_index("c")
        pltpu.sync_copy(idx_hbm.at[wid], idx_v)
        pltpu.async_copy(feats_hbm.at[idx_v], fbuf, sem).wait()
        pltpu.sync_copy(fbuf, cf_hbm.at[pl.ds(wid * GRP, GRP)])

    return k


def _silu(x):
    return x * jax.nn.sigmoid(x)


def _ln(x, g, b):
    mu = jnp.mean(x, axis=-1, keepdims=True)
    d = x - mu
    var = jnp.mean(d * d, axis=-1, keepdims=True)
    return d / jnp.sqrt(var + 1e-5) * g + b


def _mlp_body(nf_ref, nx_ref, cf_ref, cx_ref, w1_ref, w2, wp,
              b1r, g1r, be1r, b2r, g2r, be2r, bpr, out_ref, acc):
    k = pl.program_id(1)
    kk = pl.num_programs(1)

    @pl.when(k == 0)
    def _():
        acc[...] = jnp.zeros_like(acc)

    cf = cf_ref[...]
    fuse = jnp.concatenate(
        [cf, nf_ref[0] - cf, nx_ref[0] - cx_ref[...]], axis=1)
    h = _silu(jnp.dot(fuse, w1_ref[...],
                      preferred_element_type=jnp.float32) + b1r[...])
    h = _ln(h, g1r[...], be1r[...])
    h = _silu(jnp.dot(h, w2[...],
                      preferred_element_type=jnp.float32) + b2r[...])
    h = _ln(h, g2r[...], be2r[...])
    acc[...] += h

    @pl.when(k == kk - 1)
    def _():
        out_ref[...] = jnp.dot(acc[...] * (1.0 / kk), wp[...],
                               preferred_element_type=jnp.float32) + bpr[...]


def _mlp(nf3, nx3, cf, cx16, w1_144, w2, wp,
         b1, g1, be1, b2, g2, be2, bp, gb=256):
    kk, g, f = nf3.shape
    out = wp.shape[1]
    ng = g // gb
    vec = pl.BlockSpec((1, out), lambda gi, ki: (0, 0))
    sq = pl.BlockSpec((f, out), lambda gi, ki: (0, 0))
    w1s = pl.BlockSpec((f + f + 16, out), lambda gi, ki: (0, 0))
    return pl.pallas_call(
        _mlp_body,
        grid=(ng, kk),
        in_specs=[
            pl.BlockSpec((1, gb, f), lambda gi, ki: (ki, gi, 0)),
            pl.BlockSpec((1, gb, 16), lambda gi, ki: (ki, gi, 0)),
            pl.BlockSpec((gb, f), lambda gi, ki: (gi, 0)),
            pl.BlockSpec((gb, 16), lambda gi, ki: (gi, 0)),
            w1s, sq, sq,
            vec, vec, vec, vec, vec, vec, vec,
        ],
        out_specs=pl.BlockSpec((gb, out), lambda gi, ki: (gi, 0)),
        out_shape=jax.ShapeDtypeStruct((g, out), jnp.float32),
        scratch_shapes=[
            pltpu.VMEM((gb, out), jnp.float32),
        ],
        compiler_params=pltpu.CompilerParams(
            dimension_semantics=("arbitrary", "arbitrary")),
    )(nf3, nx3, cf, cx16, w1_144, w2, wp,
      b1.reshape(1, -1), g1.reshape(1, -1), be1.reshape(1, -1),
      b2.reshape(1, -1), g2.reshape(1, -1), be2.reshape(1, -1),
      bp.reshape(1, -1))


def kernel(centers_xyz, feats, points_xyz, knn_idx,
           W1, b1, g1, be1, W2, b2, g2, be2, Wp, bp):
    bsz, m, _ = centers_xyz.shape
    n, f = feats.shape[1], feats.shape[2]
    k = knn_idx.shape[2]
    out = Wp.shape[1]
    g = bsz * m
    r = g * k

    knn = knn_idx.astype(jnp.int32)
    boff = (jnp.arange(bsz, dtype=jnp.int32) * n)[:, None, None]
    gidx = (knn + boff).reshape(g, k).T.reshape(r // GRP, GRP)
    feats_flat = feats.reshape(bsz * n, f)
    xyz_flat = points_xyz.reshape(bsz * n, 3)
    xyz16 = jnp.concatenate(
        [xyz_flat, jnp.zeros((bsz * n, 13), jnp.float32)], axis=1)
    pzt = points_xyz.transpose(0, 2, 1)
    cx16 = jnp.concatenate(
        [centers_xyz.reshape(g, 3), jnp.zeros((g, 13), jnp.float32)], axis=1)
    w1_144 = jnp.concatenate(
        [W1, jnp.zeros((13, out), jnp.float32)], axis=0)

    nf_flat, nx_flat = _make_sc_gather_pair(r, f, chunk=1024)(
        gidx, feats_flat, xyz16)

    nn_rows = _nn_argmin(centers_xyz, pzt).reshape(g // GRP, GRP)

    cf = _make_sc_gather_center(g, f)(nn_rows, feats_flat)

    h = _mlp(nf_flat.reshape(k, g, f), nx_flat.reshape(k, g, 16),
             cf, cx16, w1_144, W2, Wp,
             b1, g1, be1, b2, g2, be2, bp)
    return h.reshape(bsz, m, out)


# speedup vs baseline: 1.3856x; 1.3856x over previous
import functools

import jax
import jax.numpy as jnp
from jax import lax
from jax.experimental import pallas as pl
from jax.experimental.pallas import tpu as pltpu
from jax.experimental.pallas import tpu_sc as plsc

NWORK = 32
GRP = 128


def _argmin_body(cz_ref, pzt_ref, out_ref, rmin, ridx, hmin, hidx):
    b = pl.program_id(0)
    j = pl.program_id(1)
    nch = pl.num_programs(1)
    half = nch // 2
    nc = pzt_ref.shape[2]
    n_total = nc * nch
    m = cz_ref.shape[1]

    cz = cz_ref[0]
    pzt = pzt_ref[0]
    c2 = jnp.sum(cz * cz, axis=1, keepdims=True)
    p2 = jnp.sum(pzt * pzt, axis=0, keepdims=True)
    cp = jnp.dot(cz, pzt, preferred_element_type=jnp.float32)
    d2 = (c2 + p2) - 2.0 * cp

    cmin = jnp.min(d2, axis=1, keepdims=True)
    lane = lax.broadcasted_iota(jnp.int32, d2.shape, 1)
    big = jnp.int32(2**30)
    cidx = jnp.min(jnp.where(d2 == cmin, lane, big), axis=1, keepdims=True)
    cidx = cidx + j * nc

    cmin_b = jnp.broadcast_to(cmin, (m, 128))
    cidx_b = jnp.broadcast_to(cidx, (m, 128))

    @pl.when((j == 0) | (j == half))
    def _():
        rmin[...] = cmin_b
        ridx[...] = cidx_b

    @pl.when((j != 0) & (j != half))
    def _():
        better = cmin_b < rmin[...]
        rmin[...] = jnp.where(better, cmin_b, rmin[...])
        ridx[...] = jnp.where(better, cidx_b, ridx[...])

    @pl.when(j == half - 1)
    def _():
        hmin[...] = rmin[...]
        hidx[...] = ridx[...]

    @pl.when(j == nch - 1)
    def _():
        h1b = hmin[...].astype(jnp.bfloat16).astype(jnp.float32)
        steal = rmin[...] < h1b
        nn = jnp.where(steal, ridx[...], hidx[...])
        out_ref[0] = nn[:, 0:1] + b * n_total


def _nn_argmin(centers_xyz, pzt, nc=2048):
    bsz, m, _ = centers_xyz.shape
    n = pzt.shape[2]
    grid = (bsz, n // nc)
    return pl.pallas_call(
        _argmin_body,
        grid=grid,
        in_specs=[
            pl.BlockSpec((1, m, 3), lambda b, j: (b, 0, 0)),
            pl.BlockSpec((1, 3, nc), lambda b, j: (b, 0, j)),
        ],
        out_specs=pl.BlockSpec((1, m, 1), lambda b, j: (b, 0, 0)),
        out_shape=jax.ShapeDtypeStruct((bsz, m, 1), jnp.int32),
        scratch_shapes=[
            pltpu.VMEM((m, 128), jnp.float32),
            pltpu.VMEM((m, 128), jnp.int32),
            pltpu.VMEM((m, 128), jnp.float32),
            pltpu.VMEM((m, 128), jnp.int32),
        ],
        compiler_params=pltpu.CompilerParams(
            dimension_semantics=("arbitrary", "arbitrary")),
    )(centers_xyz, pzt)


def _make_sc_gather_pair(r, f, chunk):
    ch = r // NWORK
    nt = ch // chunk
    ng = chunk // GRP
    mesh = plsc.VectorSubcoreMesh(core_axis_name="c", subcore_axis_name="s")

    @functools.partial(
        pl.kernel,
        out_type=(jax.ShapeDtypeStruct((r, f), jnp.float32),
                  jax.ShapeDtypeStruct((r, 16), jnp.float32)),
        mesh=mesh,
        scratch_types=[
            pltpu.VMEM((ch // GRP, GRP), jnp.int32),
            pltpu.VMEM((chunk, f), jnp.float32),
            pltpu.VMEM((chunk, 16), jnp.float32),
            pltpu.SemaphoreType.DMA,
            pltpu.SemaphoreType.DMA,
        ],
        compiler_params=pltpu.CompilerParams(use_tc_tiling_on_sc=False),
    )
    def k(idx_hbm, feats_hbm, xyz16_hbm, nf_hbm, nx_hbm,
          idx_v, fbuf, xbuf, semf, semx):
        wid = lax.axis_index("s") * 2 + lax.axis_index("c")
        base = wid * ch
        pltpu.sync_copy(idx_hbm.at[pl.ds(wid * (ch // GRP), ch // GRP)], idx_v)
        for t in range(nt):
            cps = []
            for g in range(ng):
                row = t * ng + g
                cps.append(pltpu.async_copy(
                    feats_hbm.at[idx_v.at[row]],
                    fbuf.at[pl.ds(g * GRP, GRP)], semf))
                cps.append(pltpu.async_copy(
                    xyz16_hbm.at[idx_v.at[row]],
                    xbuf.at[pl.ds(g * GRP, GRP)], semx))
            for c in cps:
                c.wait()
            pltpu.sync_copy(fbuf, nf_hbm.at[pl.ds(base + t * chunk, chunk)])
            pltpu.sync_copy(xbuf, nx_hbm.at[pl.ds(base + t * chunk, chunk)])

    return k


def _make_sc_gather_center(g, f):
    ch = g // NWORK
    assert ch == GRP
    mesh = plsc.VectorSubcoreMesh(core_axis_name="c", subcore_axis_name="s")

    @functools.partial(
        pl.kernel,
        out_type=jax.ShapeDtypeStruct((g, f), jnp.float32),
        mesh=mesh,
        scratch_types=[
            pltpu.VMEM((GRP,), jnp.int32),
            pltpu.VMEM((GRP, f), jnp.float32),
            pltpu.SemaphoreType.DMA,
        ],
        compiler_params=pltpu.CompilerParams(use_tc_tiling_on_sc=False),
    )
    def k(idx_hbm, feats_hbm, cf_hbm, idx_v, fbuf, sem):
        wid = lax.axis_index("s") * 2 + lax.axis_index("c")
        pltpu.sync_copy(idx_hbm.at[wid], idx_v)
        pltpu.async_copy(feats_hbm.at[idx_v], fbuf, sem).wait()
        pltpu.sync_copy(fbuf, cf_hbm.at[pl.ds(wid * GRP, GRP)])

    return k


def _silu(x):
    return x * jax.nn.sigmoid(x)


def _ln(x, g, b):
    mu = jnp.mean(x, axis=-1, keepdims=True)
    d = x - mu
    var = jnp.mean(d * d, axis=-1, keepdims=True)
    return d / jnp.sqrt(var + 1e-5) * g + b


def _mlp_body(nf_ref, nx_ref, cf_ref, cx_ref, w1_ref, w2, wp,
              b1r, g1r, be1r, b2r, g2r, be2r, bpr, out_ref, acc):
    k = pl.program_id(1)
    kk = pl.num_programs(1)

    @pl.when(k == 0)
    def _():
        acc[...] = jnp.zeros_like(acc)

    cf = cf_ref[...]
    fuse = jnp.concatenate(
        [cf, nf_ref[0] - cf, nx_ref[0] - cx_ref[...]], axis=1)
    h = _silu(jnp.dot(fuse, w1_ref[...],
                      preferred_element_type=jnp.float32) + b1r[...])
    h = _ln(h, g1r[...], be1r[...])
    h = _silu(jnp.dot(h, w2[...],
                      preferred_element_type=jnp.float32) + b2r[...])
    h = _ln(h, g2r[...], be2r[...])
    acc[...] += h

    @pl.when(k == kk - 1)
    def _():
        out_ref[...] = jnp.dot(acc[...] * (1.0 / kk), wp[...],
                               preferred_element_type=jnp.float32) + bpr[...]


def _mlp(nf3, nx3, cf, cx16, w1_144, w2, wp,
         b1, g1, be1, b2, g2, be2, bp, gb=512):
    kk, g, f = nf3.shape
    out = wp.shape[1]
    ng = g // gb
    vec = pl.BlockSpec((1, out), lambda gi, ki: (0, 0))
    sq = pl.BlockSpec((f, out), lambda gi, ki: (0, 0))
    w1s = pl.BlockSpec((f + f + 16, out), lambda gi, ki: (0, 0))
    return pl.pallas_call(
        _mlp_body,
        grid=(ng, kk),
        in_specs=[
            pl.BlockSpec((1, gb, f), lambda gi, ki: (ki, gi, 0)),
            pl.BlockSpec((1, gb, 16), lambda gi, ki: (ki, gi, 0)),
            pl.BlockSpec((gb, f), lambda gi, ki: (gi, 0)),
            pl.BlockSpec((gb, 16), lambda gi, ki: (gi, 0)),
            w1s, sq, sq,
            vec, vec, vec, vec, vec, vec, vec,
        ],
        out_specs=pl.BlockSpec((gb, out), lambda gi, ki: (gi, 0)),
        out_shape=jax.ShapeDtypeStruct((g, out), jnp.float32),
        scratch_shapes=[
            pltpu.VMEM((gb, out), jnp.float32),
        ],
        compiler_params=pltpu.CompilerParams(
            dimension_semantics=("arbitrary", "arbitrary")),
    )(nf3, nx3, cf, cx16, w1_144, w2, wp,
      b1.reshape(1, -1), g1.reshape(1, -1), be1.reshape(1, -1),
      b2.reshape(1, -1), g2.reshape(1, -1), be2.reshape(1, -1),
      bp.reshape(1, -1))


def kernel(centers_xyz, feats, points_xyz, knn_idx,
           W1, b1, g1, be1, W2, b2, g2, be2, Wp, bp):
    bsz, m, _ = centers_xyz.shape
    n, f = feats.shape[1], feats.shape[2]
    k = knn_idx.shape[2]
    out = Wp.shape[1]
    g = bsz * m
    r = g * k

    knn = knn_idx.astype(jnp.int32)
    boff = (jnp.arange(bsz, dtype=jnp.int32) * n)[:, None, None]
    gidx = (knn + boff).reshape(g, k).T.reshape(r // GRP, GRP)
    feats_flat = feats.reshape(bsz * n, f)
    xyz_flat = points_xyz.reshape(bsz * n, 3)
    xyz16 = jnp.concatenate(
        [xyz_flat, jnp.zeros((bsz * n, 13), jnp.float32)], axis=1)
    pzt = points_xyz.transpose(0, 2, 1)
    cx16 = jnp.concatenate(
        [centers_xyz.reshape(g, 3), jnp.zeros((g, 13), jnp.float32)], axis=1)
    w1_144 = jnp.concatenate(
        [W1, jnp.zeros((13, out), jnp.float32)], axis=0)

    nf_flat, nx_flat = _make_sc_gather_pair(r, f, chunk=1024)(
        gidx, feats_flat, xyz16)

    nn_rows = _nn_argmin(centers_xyz, pzt).reshape(g // GRP, GRP)

    cf = _make_sc_gather_center(g, f)(nn_rows, feats_flat)

    h = _mlp(nf_flat.reshape(k, g, f), nx_flat.reshape(k, g, 16),
             cf, cx16, w1_144, W2, Wp,
             b1, g1, be1, b2, g2, be2, bp)
    return h.reshape(bsz, m, out)


# speedup vs baseline: 1.6233x; 1.1715x over previous
import functools

import jax
import jax.numpy as jnp
from jax import lax
from jax.experimental import pallas as pl
from jax.experimental.pallas import tpu as pltpu
from jax.experimental.pallas import tpu_sc as plsc

NWORK = 32
GRP = 128


def _argmin_body(cz_ref, pzt_ref, out_ref, rmin, ridx, hmin, hidx):
    b = pl.program_id(0)
    j = pl.program_id(1)
    nch = pl.num_programs(1)
    half = nch // 2
    nc = pzt_ref.shape[2]
    n_total = nc * nch
    m = cz_ref.shape[1]

    cz = cz_ref[0]
    pzt = pzt_ref[0]
    c2 = jnp.sum(cz * cz, axis=1, keepdims=True)
    p2 = jnp.sum(pzt * pzt, axis=0, keepdims=True)
    cp = jnp.dot(cz, pzt, preferred_element_type=jnp.float32)
    d2 = (c2 + p2) - 2.0 * cp

    cmin = jnp.min(d2, axis=1, keepdims=True)
    lane = lax.broadcasted_iota(jnp.int32, d2.shape, 1)
    big = jnp.int32(2**30)
    cidx = jnp.min(jnp.where(d2 == cmin, lane, big), axis=1, keepdims=True)
    cidx = cidx + j * nc

    cmin_b = jnp.broadcast_to(cmin, (m, 128))
    cidx_b = jnp.broadcast_to(cidx, (m, 128))

    @pl.when((j == 0) | (j == half))
    def _():
        rmin[...] = cmin_b
        ridx[...] = cidx_b

    @pl.when((j != 0) & (j != half))
    def _():
        better = cmin_b < rmin[...]
        rmin[...] = jnp.where(better, cmin_b, rmin[...])
        ridx[...] = jnp.where(better, cidx_b, ridx[...])

    @pl.when(j == half - 1)
    def _():
        hmin[...] = rmin[...]
        hidx[...] = ridx[...]

    @pl.when(j == nch - 1)
    def _():
        h1b = hmin[...].astype(jnp.bfloat16).astype(jnp.float32)
        steal = rmin[...] < h1b
        nn = jnp.where(steal, ridx[...], hidx[...])
        out_ref[0] = nn[:, 0:1] + b * n_total


def _nn_argmin(centers_xyz, pzt, nc=4096):
    bsz, m, _ = centers_xyz.shape
    n = pzt.shape[2]
    grid = (bsz, n // nc)
    return pl.pallas_call(
        _argmin_body,
        grid=grid,
        in_specs=[
            pl.BlockSpec((1, m, 3), lambda b, j: (b, 0, 0)),
            pl.BlockSpec((1, 3, nc), lambda b, j: (b, 0, j)),
        ],
        out_specs=pl.BlockSpec((1, m, 1), lambda b, j: (b, 0, 0)),
        out_shape=jax.ShapeDtypeStruct((bsz, m, 1), jnp.int32),
        scratch_shapes=[
            pltpu.VMEM((m, 128), jnp.float32),
            pltpu.VMEM((m, 128), jnp.int32),
            pltpu.VMEM((m, 128), jnp.float32),
            pltpu.VMEM((m, 128), jnp.int32),
        ],
        compiler_params=pltpu.CompilerParams(
            dimension_semantics=("arbitrary", "arbitrary")),
    )(centers_xyz, pzt)


def _make_sc_gather_pair(r, f, chunk):
    ch = r // NWORK
    nt = ch // chunk
    ng = chunk // GRP
    mesh = plsc.VectorSubcoreMesh(core_axis_name="c", subcore_axis_name="s")

    @functools.partial(
        pl.kernel,
        out_type=(jax.ShapeDtypeStruct((r, f), jnp.float32),
                  jax.ShapeDtypeStruct((r, 16), jnp.float32)),
        mesh=mesh,
        scratch_types=[
            pltpu.VMEM((ch // GRP, GRP), jnp.int32),
            pltpu.VMEM((chunk, f), jnp.float32),
            pltpu.VMEM((chunk, 16), jnp.float32),
            pltpu.SemaphoreType.DMA,
            pltpu.SemaphoreType.DMA,
        ],
        compiler_params=pltpu.CompilerParams(use_tc_tiling_on_sc=False),
    )
    def k(idx_hbm, feats_hbm, xyz16_hbm, nf_hbm, nx_hbm,
          idx_v, fbuf, xbuf, semf, semx):
        wid = lax.axis_index("s") * 2 + lax.axis_index("c")
        base = wid * ch
        pltpu.sync_copy(idx_hbm.at[pl.ds(wid * (ch // GRP), ch // GRP)], idx_v)
        for t in range(nt):
            cps = []
            for g in range(ng):
                row = t * ng + g
                cps.append(pltpu.async_copy(
                    feats_hbm.at[idx_v.at[row]],
                    fbuf.at[pl.ds(g * GRP, GRP)], semf))
                cps.append(pltpu.async_copy(
                    xyz16_hbm.at[idx_v.at[row]],
                    xbuf.at[pl.ds(g * GRP, GRP)], semx))
            for c in cps:
                c.wait()
            pltpu.sync_copy(fbuf, nf_hbm.at[pl.ds(base + t * chunk, chunk)])
            pltpu.sync_copy(xbuf, nx_hbm.at[pl.ds(base + t * chunk, chunk)])

    return k


def _make_sc_gather_center(g, f):
    ch = g // NWORK
    assert ch == GRP
    mesh = plsc.VectorSubcoreMesh(core_axis_name="c", subcore_axis_name="s")

    @functools.partial(
        pl.kernel,
        out_type=jax.ShapeDtypeStruct((g, f), jnp.float32),
        mesh=mesh,
        scratch_types=[
            pltpu.VMEM((GRP,), jnp.int32),
            pltpu.VMEM((GRP, f), jnp.float32),
            pltpu.SemaphoreType.DMA,
        ],
        compiler_params=pltpu.CompilerParams(use_tc_tiling_on_sc=False),
    )
    def k(idx_hbm, feats_hbm, cf_hbm, idx_v, fbuf, sem):
        wid = lax.axis_index("s") * 2 + lax.axis_index("c")
        pltpu.sync_copy(idx_hbm.at[wid], idx_v)
        pltpu.async_copy(feats_hbm.at[idx_v], fbuf, sem).wait()
        pltpu.sync_copy(fbuf, cf_hbm.at[pl.ds(wid * GRP, GRP)])

    return k


def _silu(x):
    return x * jax.nn.sigmoid(x)


def _ln(x, g, b):
    mu = jnp.mean(x, axis=-1, keepdims=True)
    d = x - mu
    var = jnp.mean(d * d, axis=-1, keepdims=True)
    return d / jnp.sqrt(var + 1e-5) * g + b


def _mlp_body(nf_ref, nx_ref, cf_ref, cx_ref, w1_ref, w2, wp,
              b1r, g1r, be1r, b2r, g2r, be2r, bpr, out_ref, acc):
    k = pl.program_id(1)
    kk = pl.num_programs(1)

    @pl.when(k == 0)
    def _():
        acc[...] = jnp.zeros_like(acc)

    cf = cf_ref[...]
    fuse = jnp.concatenate(
        [cf, nf_ref[0] - cf, nx_ref[0] - cx_ref[...]], axis=1)
    h = _silu(jnp.dot(fuse, w1_ref[...],
                      preferred_element_type=jnp.float32) + b1r[...])
    h = _ln(h, g1r[...], be1r[...])
    h = _silu(jnp.dot(h, w2[...],
                      preferred_element_type=jnp.float32) + b2r[...])
    h = _ln(h, g2r[...], be2r[...])
    acc[...] += h

    @pl.when(k == kk - 1)
    def _():
        out_ref[...] = jnp.dot(acc[...] * (1.0 / kk), wp[...],
                               preferred_element_type=jnp.float32) + bpr[...]


def _mlp(nf3, nx3, cf, cx16, w1_144, w2, wp,
         b1, g1, be1, b2, g2, be2, bp, gb=1024):
    kk, g, f = nf3.shape
    out = wp.shape[1]
    ng = g // gb
    vec = pl.BlockSpec((1, out), lambda gi, ki: (0, 0))
    sq = pl.BlockSpec((f, out), lambda gi, ki: (0, 0))
    w1s = pl.BlockSpec((f + f + 16, out), lambda gi, ki: (0, 0))
    return pl.pallas_call(
        _mlp_body,
        grid=(ng, kk),
        in_specs=[
            pl.BlockSpec((1, gb, f), lambda gi, ki: (ki, gi, 0)),
            pl.BlockSpec((1, gb, 16), lambda gi, ki: (ki, gi, 0)),
            pl.BlockSpec((gb, f), lambda gi, ki: (gi, 0)),
            pl.BlockSpec((gb, 16), lambda gi, ki: (gi, 0)),
            w1s, sq, sq,
            vec, vec, vec, vec, vec, vec, vec,
        ],
        out_specs=pl.BlockSpec((gb, out), lambda gi, ki: (gi, 0)),
        out_shape=jax.ShapeDtypeStruct((g, out), jnp.float32),
        scratch_shapes=[
            pltpu.VMEM((gb, out), jnp.float32),
        ],
        compiler_params=pltpu.CompilerParams(
            dimension_semantics=("arbitrary", "arbitrary")),
    )(nf3, nx3, cf, cx16, w1_144, w2, wp,
      b1.reshape(1, -1), g1.reshape(1, -1), be1.reshape(1, -1),
      b2.reshape(1, -1), g2.reshape(1, -1), be2.reshape(1, -1),
      bp.reshape(1, -1))


def kernel(centers_xyz, feats, points_xyz, knn_idx,
           W1, b1, g1, be1, W2, b2, g2, be2, Wp, bp):
    bsz, m, _ = centers_xyz.shape
    n, f = feats.shape[1], feats.shape[2]
    k = knn_idx.shape[2]
    out = Wp.shape[1]
    g = bsz * m
    r = g * k

    knn = knn_idx.astype(jnp.int32)
    boff = (jnp.arange(bsz, dtype=jnp.int32) * n)[:, None, None]
    gidx = (knn + boff).reshape(g, k).T.reshape(r // GRP, GRP)
    feats_flat = feats.reshape(bsz * n, f)
    xyz_flat = points_xyz.reshape(bsz * n, 3)
    xyz16 = jnp.concatenate(
        [xyz_flat, jnp.zeros((bsz * n, 13), jnp.float32)], axis=1)
    pzt = points_xyz.transpose(0, 2, 1)
    cx16 = jnp.concatenate(
        [centers_xyz.reshape(g, 3), jnp.zeros((g, 13), jnp.float32)], axis=1)
    w1_144 = jnp.concatenate(
        [W1, jnp.zeros((13, out), jnp.float32)], axis=0)

    nf_flat, nx_flat = _make_sc_gather_pair(r, f, chunk=1024)(
        gidx, feats_flat, xyz16)

    nn_rows = _nn_argmin(centers_xyz, pzt).reshape(g // GRP, GRP)

    cf = _make_sc_gather_center(g, f)(nn_rows, feats_flat)

    h = _mlp(nf_flat.reshape(k, g, f), nx_flat.reshape(k, g, 16),
             cf, cx16, w1_144, W2, Wp,
             b1, g1, be1, b2, g2, be2, bp)
    return h.reshape(bsz, m, out)
